# Initial kernel scaffold; baseline (speedup 1.0000x reference)
import jax
import jax.numpy as jnp
from jax.experimental import pallas as pl


def kernel(x, edge_index, batch, head_W, head_b, bn_g, bn_b, s0_W, s0_b, sc1_W, sc1_b, sl1_W, sl1_b, st_W, st_b, i0_W, i0_b, ic1_W, ic1_b, il1_W, il1_b, ic2_W, ic2_b, il2_W, il2_b, it_W, it_b, cls_W, cls_b):
    raise NotImplementedError("write your pallas kernel here")


# speedup vs baseline: 10.7966x; 10.7966x over previous
import functools

import jax
import jax.numpy as jnp
from jax import lax
from jax.experimental import pallas as pl
from jax.experimental.pallas import tpu as pltpu
from jax.experimental.pallas import tpu_sc as plsc

_N = 10000
_E = 320000
_DIN = 128
_H = 64
_DT = 4
_NB = 8

_BN_SCALE = 1.0 / (1.0 + 1e-5) ** 0.5

_NC = 2
_NS = 16
_NW = _NC * _NS
_CHUNK = 80
_NT = _E // _CHUNK
_TPW = _NT // _NW
_GRP = 5
_NGRP = _TPW // _GRP
_NPAD = 10240
_RPT = _NPAD // _NS

@functools.cache
def _get_sc_agg():
    mesh = plsc.VectorSubcoreMesh(core_axis_name="c", subcore_axis_name="s",
                                  num_cores=_NC, num_subcores=_NS)
    return functools.partial(
        pl.kernel,
        out_type=jax.ShapeDtypeStruct((_NC * _NPAD, _H), jnp.float32),
        mesh=mesh,
        scratch_types=[
            pltpu.VMEM((_TPW, _CHUNK), jnp.int32),
            pltpu.VMEM((_TPW, _CHUNK), jnp.int32),
            pltpu.VMEM((_GRP, _CHUNK, _H), jnp.float32),
            pltpu.VMEM_SHARED((_NPAD, _H), jnp.float32),
            pltpu.SemaphoreType.DMA,
            pltpu.SemaphoreType.DMA,
        ],
        compiler_params=pltpu.CompilerParams(use_tc_tiling_on_sc=False),
    )(_sc_agg_body)


def _sc_agg_body(h_hbm, src_hbm, dst_hbm, zeros_hbm, out_hbm,
                 src_v, dst_v, rows_v, acc_sh, sem_ld, sem_st):
    c = lax.axis_index("c")
    s = lax.axis_index("s")
    wid = s * _NC + c

    pltpu.sync_copy(zeros_hbm, acc_sh.at[pl.ds(s * _RPT, _RPT)])
    d_src = pltpu.async_copy(src_hbm.at[wid], src_v, sem_ld)
    d_dst = pltpu.async_copy(dst_hbm.at[wid], dst_v, sem_ld)
    d_src.wait()
    d_dst.wait()
    plsc.subcore_barrier()

    def group(gi, carry):
        gathers = []
        for j in range(_GRP):
            t = gi * _GRP + j
            gathers.append(
                pltpu.async_copy(h_hbm.at[src_v.at[t]], rows_v.at[j], sem_ld))
        for d in gathers:
            d.wait()
        scatters = []
        for j in range(_GRP):
            t = gi * _GRP + j
            scatters.append(
                pltpu.async_copy(rows_v.at[j], acc_sh.at[dst_v.at[t]],
                                 sem_st, add=True))
        for d in scatters:
            d.wait()
        return carry

    lax.fori_loop(0, _NGRP, group, 0)
    plsc.subcore_barrier()
    pltpu.sync_copy(acc_sh.at[pl.ds(s * _RPT, _RPT)],
                    out_hbm.at[pl.ds(c * _NPAD + s * _RPT, _RPT)])


_B = 1000
_G = _N // _B


def _dot(a, b):
    return jnp.dot(a, b, preferred_element_type=jnp.float32)


def _tc1_body(x_ref, hw_ref, hb_ref, g_ref, b_ref, s0w_ref, s0b_ref,
              f_ref, h0_ref):
    y = _dot(x_ref[...], hw_ref[...]) + hb_ref[...]
    f = jnp.maximum(y * (g_ref[...] * _BN_SCALE) + b_ref[...], 0.0)
    f_ref[...] = f
    h0_ref[...] = jnp.maximum(_dot(f, s0w_ref[...]) + s0b_ref[...], 0.0)


def _tc2_body(h0_ref, agg_ref, f_ref, eps_ref,
              c1w_ref, c1b_ref, l1w_ref, l1b_ref, stw_ref, stb_ref,
              i0w_ref, i0b_ref, d0_ref, g_out_ref):
    h = h0_ref[...] + agg_ref[0] + agg_ref[1]
    h = jnp.maximum(_dot(h, c1w_ref[...]) + c1b_ref[...], 0.0)
    h = _dot(h, l1w_ref[...]) + l1b_ref[...]
    enc = _dot(h, stw_ref[...]) + stb_ref[...]
    loc = enc[:, 0:1]
    logvar = jnp.clip(enc[:, 1:2], -20.0, 20.0)
    gate = loc + jnp.exp(0.5 * logvar) * eps_ref[...]
    g_out_ref[...] = f_ref[...] * gate
    d0_ref[...] = jnp.maximum(_dot(enc, i0w_ref[...]) + i0b_ref[...], 0.0)


def _tc3_body(d0_ref, agg_ref, c1w_ref, c1b_ref, l1w_ref, l1b_ref, d2_ref):
    d = d0_ref[...] + agg_ref[0] + agg_ref[1]
    d = jnp.maximum(_dot(d, c1w_ref[...]) + c1b_ref[...], 0.0)
    d2_ref[...] = _dot(d, l1w_ref[...]) + l1b_ref[...]


def _tc4_body(d2_ref, agg_ref, g_ref, batch_ref,
              c2w_ref, c2b_ref, l2w_ref, l2b_ref, itw_ref, itb_ref,
              clsw_ref, clsb_ref, dec_ref, out_ref, pool_s, cnt_s):
    i = pl.program_id(0)
    d = d2_ref[...] + agg_ref[0] + agg_ref[1]
    d = jnp.maximum(_dot(d, c2w_ref[...]) + c2b_ref[...], 0.0)
    d = _dot(d, l2w_ref[...]) + l2b_ref[...]
    dec_ref[...] = _dot(d, itw_ref[...]) + itb_ref[...]

    onehot = (batch_ref[...] ==
              lax.broadcasted_iota(jnp.int32, (_B, _NB), 1)).astype(jnp.float32)
    dims = (((0,), (0,)), ((), ()))
    psum = lax.dot_general(onehot, g_ref[...], dims,
                           preferred_element_type=jnp.float32)
    csum = lax.dot_general(onehot, jnp.ones((_B, 1), jnp.float32), dims,
                           preferred_element_type=jnp.float32)

    @pl.when(i == 0)
    def _():
        pool_s[...] = jnp.zeros_like(pool_s)
        cnt_s[...] = jnp.zeros_like(cnt_s)

    pool_s[...] += psum
    cnt_s[...] += csum

    @pl.when(i == pl.num_programs(0) - 1)
    def _():
        pooled = pool_s[...] / jnp.maximum(cnt_s[...], 1.0)
        out_ref[...] = _dot(pooled, clsw_ref[...]) + clsb_ref[...]


def _full(shape):
    return pl.BlockSpec(shape, lambda i: tuple(0 for _ in shape))


def _rows(w):
    return pl.BlockSpec((_B, w), lambda i: (i, 0))


_agg_spec = pl.BlockSpec((2, _B, _H), lambda i: (0, i, 0))

_tc1 = pl.pallas_call(
    _tc1_body,
    grid=(_G,),
    in_specs=[_rows(_DIN), _full((_DIN, _H)), _full((1, _H)), _full((1, _H)),
              _full((1, _H)), _full((_H, _H)), _full((1, _H))],
    out_specs=[_rows(_H), _rows(_H)],
    out_shape=[jax.ShapeDtypeStruct((_N, _H), jnp.float32)] * 2,
)

_tc2 = pl.pallas_call(
    _tc2_body,
    grid=(_G,),
    in_specs=[_rows(_H), _agg_spec, _rows(_H), _rows(1),
              _full((_H, _H)), _full((1, _H)), _full((_H, _H)), _full((1, _H)),
              _full((_H, 2)), _full((1, 2)), _full((2, _H)), _full((1, _H))],
    out_specs=[_rows(_H), _rows(_H)],
    out_shape=[jax.ShapeDtypeStruct((_N, _H), jnp.float32)] * 2,
)

_tc3 = pl.pallas_call(
    _tc3_body,
    grid=(_G,),
    in_specs=[_rows(_H), _agg_spec,
              _full((_H, _H)), _full((1, _H)), _full((_H, _H)), _full((1, _H))],
    out_specs=[_rows(_H)],
    out_shape=[jax.ShapeDtypeStruct((_N, _H), jnp.float32)],
)

_tc4 = pl.pallas_call(
    _tc4_body,
    grid=(_G,),
    in_specs=[_rows(_H), _agg_spec, _rows(_H), _rows(1),
              _full((_H, _H)), _full((1, _H)), _full((_H, _H)), _full((1, _H)),
              _full((_H, _H)), _full((1, _H)), _full((_H, _DT)), _full((1, _DT))],
    out_specs=[_rows(_H), _full((_NB, _DT))],
    out_shape=[jax.ShapeDtypeStruct((_N, _H), jnp.float32),
               jax.ShapeDtypeStruct((_NB, _DT), jnp.float32)],
    scratch_shapes=[pltpu.VMEM((_NB, _H), jnp.float32),
                    pltpu.VMEM((_NB, 1), jnp.float32)],
)


def kernel(x, edge_index, batch, head_W, head_b, bn_g, bn_b, s0_W, s0_b,
           sc1_W, sc1_b, sl1_W, sl1_b, st_W, st_b, i0_W, i0_b,
           ic1_W, ic1_b, il1_W, il1_b, ic2_W, ic2_b, il2_W, il2_b,
           it_W, it_b, cls_W, cls_b):
    src = edge_index[0].reshape(_NW, _TPW, _CHUNK)
    dst = edge_index[1].reshape(_NW, _TPW, _CHUNK)
    zeros = jnp.zeros((_RPT, _H), jnp.float32)
    eps = jax.random.normal(jax.random.key(42), (_N, 1), dtype=jnp.float32)
    batch2 = batch.reshape(_N, 1)
    r = lambda v: v.reshape(1, -1)

    f, h0 = _tc1(x, head_W, r(head_b), r(bn_g), r(bn_b), s0_W, r(s0_b))
    sc_agg = _get_sc_agg()
    agg1 = sc_agg(h0, src, dst, zeros).reshape(2, _NPAD, _H)
    d0, g = _tc2(h0, agg1, f, eps, sc1_W, r(sc1_b), sl1_W, r(sl1_b),
                 st_W, r(st_b), i0_W, r(i0_b))
    agg2 = sc_agg(d0, src, dst, zeros).reshape(2, _NPAD, _H)
    (d2,) = _tc3(d0, agg2, ic1_W, r(ic1_b), il1_W, r(il1_b))
    agg3 = sc_agg(d2, src, dst, zeros).reshape(2, _NPAD, _H)
    decode, out = _tc4(d2, agg3, g, batch2, ic2_W, r(ic2_b), il2_W, r(il2_b),
                       it_W, r(it_b), cls_W, r(cls_b))
    return out, decode


# speedup vs baseline: 11.2942x; 1.0461x over previous
import functools

import jax
import jax.numpy as jnp
from jax import lax
from jax.experimental import pallas as pl
from jax.experimental.pallas import tpu as pltpu
from jax.experimental.pallas import tpu_sc as plsc

_N = 10000
_E = 320000
_DIN = 128
_H = 64
_DT = 4
_NB = 8

_BN_SCALE = 1.0 / (1.0 + 1e-5) ** 0.5

_NC = 2
_NS = 16
_NW = _NC * _NS
_CHUNK = 125
_NT = _E // _CHUNK
_TPW = _NT // _NW
_GRP = 4
_NGRP = _TPW // _GRP
_NPAD = 10240
_RPT = _NPAD // _NS

@functools.cache
def _get_sc_agg():
    mesh = plsc.VectorSubcoreMesh(core_axis_name="c", subcore_axis_name="s",
                                  num_cores=_NC, num_subcores=_NS)
    return functools.partial(
        pl.kernel,
        out_type=jax.ShapeDtypeStruct((_NC * _NPAD, _H), jnp.float32),
        mesh=mesh,
        scratch_types=[
            pltpu.VMEM((_TPW, _CHUNK), jnp.int32),
            pltpu.VMEM((_TPW, _CHUNK), jnp.int32),
            pltpu.VMEM((2, _GRP, _CHUNK, _H), jnp.float32),
            pltpu.VMEM_SHARED((_NPAD, _H), jnp.float32),
            pltpu.SemaphoreType.DMA,
            pltpu.SemaphoreType.DMA,
            pltpu.SemaphoreType.DMA,
            pltpu.SemaphoreType.DMA,
        ],
        compiler_params=pltpu.CompilerParams(use_tc_tiling_on_sc=False),
    )(_sc_agg_body)


def _sc_agg_body(h_hbm, src_hbm, dst_hbm, zeros_hbm, out_hbm,
                 src_v, dst_v, rows_v, acc_sh,
                 sem_ld0, sem_ld1, sem_st0, sem_st1):
    c = lax.axis_index("c")
    s = lax.axis_index("s")
    wid = s * _NC + c
    sem_ld = (sem_ld0, sem_ld1)
    sem_st = (sem_st0, sem_st1)

    pltpu.sync_copy(zeros_hbm, acc_sh.at[pl.ds(s * _RPT, _RPT)])
    d_src = pltpu.async_copy(src_hbm.at[wid], src_v, sem_ld0)
    d_dst = pltpu.async_copy(dst_hbm.at[wid], dst_v, sem_ld0)
    d_src.wait()
    d_dst.wait()
    plsc.subcore_barrier()

    def fire_gathers(g, b):
        for j in range(_GRP):
            pltpu.async_copy(h_hbm.at[src_v.at[g * _GRP + j]],
                             rows_v.at[b, j], sem_ld[b])

    def drain_gathers(g, b):
        for j in range(_GRP):
            pltpu.make_async_copy(h_hbm.at[src_v.at[g * _GRP + j]],
                                  rows_v.at[b, j], sem_ld[b]).wait()

    def fire_scatters(g, b):
        for j in range(_GRP):
            pltpu.async_copy(rows_v.at[b, j], acc_sh.at[dst_v.at[g * _GRP + j]],
                             sem_st[b], add=True)

    def drain_scatters(g, b):
        for j in range(_GRP):
            pltpu.make_async_copy(rows_v.at[b, j],
                                  acc_sh.at[dst_v.at[g * _GRP + j]],
                                  sem_st[b]).wait()

    fire_gathers(0, 0)

    def pair(i, carry):
        g0 = 2 * i
        g1 = g0 + 1

        @pl.when(i > 0)
        def _():
            drain_scatters(g1 - 2, 1)
        fire_gathers(g1, 1)
        drain_gathers(g0, 0)
        fire_scatters(g0, 0)
        drain_gathers(g1, 1)
        fire_scatters(g1, 1)

        @pl.when(i < _NGRP // 2 - 1)
        def _():
            drain_scatters(g0, 0)
            fire_gathers(g0 + 2, 0)
        return carry

    lax.fori_loop(0, _NGRP // 2, pair, 0)
    drain_scatters(_NGRP - 2, 0)
    drain_scatters(_NGRP - 1, 1)
    plsc.subcore_barrier()
    pltpu.sync_copy(acc_sh.at[pl.ds(s * _RPT, _RPT)],
                    out_hbm.at[pl.ds(c * _NPAD + s * _RPT, _RPT)])


_B = 1000
_G = _N // _B


def _dot(a, b):
    return jnp.dot(a, b, preferred_element_type=jnp.float32)


def _tc1_body(x_ref, hw_ref, hb_ref, g_ref, b_ref, s0w_ref, s0b_ref,
              f_ref, h0_ref):
    y = _dot(x_ref[...], hw_ref[...]) + hb_ref[...]
    f = jnp.maximum(y * (g_ref[...] * _BN_SCALE) + b_ref[...], 0.0)
    f_ref[...] = f
    h0_ref[...] = jnp.maximum(_dot(f, s0w_ref[...]) + s0b_ref[...], 0.0)


def _tc2_body(h0_ref, agg_ref, f_ref, eps_ref,
              c1w_ref, c1b_ref, l1w_ref, l1b_ref, stw_ref, stb_ref,
              i0w_ref, i0b_ref, d0_ref, g_out_ref):
    h = h0_ref[...] + agg_ref[0] + agg_ref[1]
    h = jnp.maximum(_dot(h, c1w_ref[...]) + c1b_ref[...], 0.0)
    h = _dot(h, l1w_ref[...]) + l1b_ref[...]
    enc = _dot(h, stw_ref[...]) + stb_ref[...]
    loc = enc[:, 0:1]
    logvar = jnp.clip(enc[:, 1:2], -20.0, 20.0)
    gate = loc + jnp.exp(0.5 * logvar) * eps_ref[...]
    g_out_ref[...] = f_ref[...] * gate
    d0_ref[...] = jnp.maximum(_dot(enc, i0w_ref[...]) + i0b_ref[...], 0.0)


def _tc3_body(d0_ref, agg_ref, c1w_ref, c1b_ref, l1w_ref, l1b_ref, d2_ref):
    d = d0_ref[...] + agg_ref[0] + agg_ref[1]
    d = jnp.maximum(_dot(d, c1w_ref[...]) + c1b_ref[...], 0.0)
    d2_ref[...] = _dot(d, l1w_ref[...]) + l1b_ref[...]


def _tc4_body(d2_ref, agg_ref, g_ref, batch_ref,
              c2w_ref, c2b_ref, l2w_ref, l2b_ref, itw_ref, itb_ref,
              clsw_ref, clsb_ref, dec_ref, out_ref, pool_s, cnt_s):
    i = pl.program_id(0)
    d = d2_ref[...] + agg_ref[0] + agg_ref[1]
    d = jnp.maximum(_dot(d, c2w_ref[...]) + c2b_ref[...], 0.0)
    d = _dot(d, l2w_ref[...]) + l2b_ref[...]
    dec_ref[...] = _dot(d, itw_ref[...]) + itb_ref[...]

    onehot = (batch_ref[...] ==
              lax.broadcasted_iota(jnp.int32, (_B, _NB), 1)).astype(jnp.float32)
    dims = (((0,), (0,)), ((), ()))
    psum = lax.dot_general(onehot, g_ref[...], dims,
                           preferred_element_type=jnp.float32)
    csum = lax.dot_general(onehot, jnp.ones((_B, 1), jnp.float32), dims,
                           preferred_element_type=jnp.float32)

    @pl.when(i == 0)
    def _():
        pool_s[...] = jnp.zeros_like(pool_s)
        cnt_s[...] = jnp.zeros_like(cnt_s)

    pool_s[...] += psum
    cnt_s[...] += csum

    @pl.when(i == pl.num_programs(0) - 1)
    def _():
        pooled = pool_s[...] / jnp.maximum(cnt_s[...], 1.0)
        out_ref[...] = _dot(pooled, clsw_ref[...]) + clsb_ref[...]


def _full(shape):
    return pl.BlockSpec(shape, lambda i: tuple(0 for _ in shape))


def _rows(w):
    return pl.BlockSpec((_B, w), lambda i: (i, 0))


_agg_spec = pl.BlockSpec((2, _B, _H), lambda i: (0, i, 0))

_tc1 = pl.pallas_call(
    _tc1_body,
    grid=(_G,),
    in_specs=[_rows(_DIN), _full((_DIN, _H)), _full((1, _H)), _full((1, _H)),
              _full((1, _H)), _full((_H, _H)), _full((1, _H))],
    out_specs=[_rows(_H), _rows(_H)],
    out_shape=[jax.ShapeDtypeStruct((_N, _H), jnp.float32)] * 2,
)

_tc2 = pl.pallas_call(
    _tc2_body,
    grid=(_G,),
    in_specs=[_rows(_H), _agg_spec, _rows(_H), _rows(1),
              _full((_H, _H)), _full((1, _H)), _full((_H, _H)), _full((1, _H)),
              _full((_H, 2)), _full((1, 2)), _full((2, _H)), _full((1, _H))],
    out_specs=[_rows(_H), _rows(_H)],
    out_shape=[jax.ShapeDtypeStruct((_N, _H), jnp.float32)] * 2,
)

_tc3 = pl.pallas_call(
    _tc3_body,
    grid=(_G,),
    in_specs=[_rows(_H), _agg_spec,
              _full((_H, _H)), _full((1, _H)), _full((_H, _H)), _full((1, _H))],
    out_specs=[_rows(_H)],
    out_shape=[jax.ShapeDtypeStruct((_N, _H), jnp.float32)],
)

_tc4 = pl.pallas_call(
    _tc4_body,
    grid=(_G,),
    in_specs=[_rows(_H), _agg_spec, _rows(_H), _rows(1),
              _full((_H, _H)), _full((1, _H)), _full((_H, _H)), _full((1, _H)),
              _full((_H, _H)), _full((1, _H)), _full((_H, _DT)), _full((1, _DT))],
    out_specs=[_rows(_H), _full((_NB, _DT))],
    out_shape=[jax.ShapeDtypeStruct((_N, _H), jnp.float32),
               jax.ShapeDtypeStruct((_NB, _DT), jnp.float32)],
    scratch_shapes=[pltpu.VMEM((_NB, _H), jnp.float32),
                    pltpu.VMEM((_NB, 1), jnp.float32)],
)


def kernel(x, edge_index, batch, head_W, head_b, bn_g, bn_b, s0_W, s0_b,
           sc1_W, sc1_b, sl1_W, sl1_b, st_W, st_b, i0_W, i0_b,
           ic1_W, ic1_b, il1_W, il1_b, ic2_W, ic2_b, il2_W, il2_b,
           it_W, it_b, cls_W, cls_b):
    src = edge_index[0].reshape(_NW, _TPW, _CHUNK)
    dst = edge_index[1].reshape(_NW, _TPW, _CHUNK)
    zeros = jnp.zeros((_RPT, _H), jnp.float32)
    eps = jax.random.normal(jax.random.key(42), (_N, 1), dtype=jnp.float32)
    batch2 = batch.reshape(_N, 1)
    r = lambda v: v.reshape(1, -1)

    f, h0 = _tc1(x, head_W, r(head_b), r(bn_g), r(bn_b), s0_W, r(s0_b))
    sc_agg = _get_sc_agg()
    agg1 = sc_agg(h0, src, dst, zeros).reshape(2, _NPAD, _H)
    d0, g = _tc2(h0, agg1, f, eps, sc1_W, r(sc1_b), sl1_W, r(sl1_b),
                 st_W, r(st_b), i0_W, r(i0_b))
    agg2 = sc_agg(d0, src, dst, zeros).reshape(2, _NPAD, _H)
    (d2,) = _tc3(d0, agg2, ic1_W, r(ic1_b), il1_W, r(il1_b))
    agg3 = sc_agg(d2, src, dst, zeros).reshape(2, _NPAD, _H)
    decode, out = _tc4(d2, agg3, g, batch2, ic2_W, r(ic2_b), il2_W, r(il2_b),
                       it_W, r(it_b), cls_W, r(cls_b))
    return out, decode


# speedup vs baseline: 12.3313x; 1.0918x over previous
import functools

import jax
import jax.numpy as jnp
import numpy as np
from jax import lax
from jax.experimental import pallas as pl
from jax.experimental.pallas import tpu as pltpu
from jax.experimental.pallas import tpu_sc as plsc

_N = 10000
_E = 320000
_DIN = 128
_H = 64
_DT = 4
_NB = 8

_BN_SCALE = 1.0 / (1.0 + 1e-5) ** 0.5

_EPS = np.asarray(
    jax.random.normal(jax.random.key(42), (_N, 1), dtype=jnp.float32))

_NC = 2
_NS = 16
_NW = _NC * _NS
_CHUNK = 125
_NT = _E // _CHUNK
_TPW = _NT // _NW
_GRP = 4
_NGRP = _TPW // _GRP
_NPAD = 10240
_RPT = _NPAD // _NS

@functools.cache
def _get_sc_agg():
    mesh = plsc.VectorSubcoreMesh(core_axis_name="c", subcore_axis_name="s",
                                  num_cores=_NC, num_subcores=_NS)
    return functools.partial(
        pl.kernel,
        out_type=jax.ShapeDtypeStruct((_NPAD, 2 * _H), jnp.float32),
        mesh=mesh,
        scratch_types=[
            pltpu.VMEM((_TPW, _CHUNK), jnp.int32),
            pltpu.VMEM((_TPW, _CHUNK), jnp.int32),
            pltpu.VMEM((2, _GRP, _CHUNK, _H), jnp.float32),
            pltpu.VMEM_SHARED((_NPAD, _H), jnp.float32),
            pltpu.SemaphoreType.DMA,
            pltpu.SemaphoreType.DMA,
            pltpu.SemaphoreType.DMA,
            pltpu.SemaphoreType.DMA,
        ],
        compiler_params=pltpu.CompilerParams(use_tc_tiling_on_sc=False),
    )(_sc_agg_body)


def _sc_agg_body(h_hbm, src_hbm, dst_hbm, zeros_hbm, out_hbm,
                 src_v, dst_v, rows_v, acc_sh,
                 sem_ld0, sem_ld1, sem_st0, sem_st1):
    c = lax.axis_index("c")
    s = lax.axis_index("s")
    wid = s * _NC + c
    sem_ld = (sem_ld0, sem_ld1)
    sem_st = (sem_st0, sem_st1)

    pltpu.sync_copy(zeros_hbm, acc_sh.at[pl.ds(s * _RPT, _RPT)])
    d_src = pltpu.async_copy(src_hbm.at[wid], src_v, sem_ld0)
    d_dst = pltpu.async_copy(dst_hbm.at[wid], dst_v, sem_ld0)
    d_src.wait()
    d_dst.wait()
    plsc.subcore_barrier()

    def fire_gathers(g, b):
        for j in range(_GRP):
            pltpu.async_copy(h_hbm.at[src_v.at[g * _GRP + j]],
                             rows_v.at[b, j], sem_ld[b])

    def drain_gathers(g, b):
        for j in range(_GRP):
            pltpu.make_async_copy(h_hbm.at[src_v.at[g * _GRP + j]],
                                  rows_v.at[b, j], sem_ld[b]).wait()

    def fire_scatters(g, b):
        for j in range(_GRP):
            pltpu.async_copy(rows_v.at[b, j], acc_sh.at[dst_v.at[g * _GRP + j]],
                             sem_st[b], add=True)

    def drain_scatters(g, b):
        for j in range(_GRP):
            pltpu.make_async_copy(rows_v.at[b, j],
                                  acc_sh.at[dst_v.at[g * _GRP + j]],
                                  sem_st[b]).wait()

    fire_gathers(0, 0)

    def pair(i, carry):
        g0 = 2 * i
        g1 = g0 + 1

        @pl.when(i > 0)
        def _():
            drain_scatters(g1 - 2, 1)
        fire_gathers(g1, 1)
        drain_gathers(g0, 0)
        fire_scatters(g0, 0)
        drain_gathers(g1, 1)
        fire_scatters(g1, 1)

        @pl.when(i < _NGRP // 2 - 1)
        def _():
            drain_scatters(g0, 0)
            fire_gathers(g0 + 2, 0)
        return carry

    lax.fori_loop(0, _NGRP // 2, pair, 0)
    drain_scatters(_NGRP - 2, 0)
    drain_scatters(_NGRP - 1, 1)
    plsc.subcore_barrier()
    pltpu.sync_copy(acc_sh.at[pl.ds(s * _RPT, _RPT)],
                    out_hbm.at[pl.ds(s * _RPT, _RPT), pl.ds(c * _H, _H)])


_B = 1000
_G = _N // _B


def _dot(a, b):
    return jnp.dot(a, b, preferred_element_type=jnp.float32)


def _tc1_body(x_ref, hw_ref, hb_ref, g_ref, b_ref, s0w_ref, s0b_ref,
              f_ref, h0_ref):
    y = _dot(x_ref[...], hw_ref[...]) + hb_ref[...]
    f = jnp.maximum(y * (g_ref[...] * _BN_SCALE) + b_ref[...], 0.0)
    f_ref[...] = f
    h0_ref[...] = jnp.maximum(_dot(f, s0w_ref[...]) + s0b_ref[...], 0.0)


def _tc2_body(h0_ref, agg_ref, f_ref, eps_ref,
              c1w_ref, c1b_ref, l1w_ref, l1b_ref, stw_ref, stb_ref,
              i0w_ref, i0b_ref, d0_ref, g_out_ref):
    h = h0_ref[...] + agg_ref[:, :_H] + agg_ref[:, _H:]
    h = jnp.maximum(_dot(h, c1w_ref[...]) + c1b_ref[...], 0.0)
    h = _dot(h, l1w_ref[...]) + l1b_ref[...]
    enc = _dot(h, stw_ref[...]) + stb_ref[...]
    loc = enc[:, 0:1]
    logvar = jnp.clip(enc[:, 1:2], -20.0, 20.0)
    gate = loc + jnp.exp(0.5 * logvar) * eps_ref[...]
    g_out_ref[...] = f_ref[...] * gate
    d0_ref[...] = jnp.maximum(_dot(enc, i0w_ref[...]) + i0b_ref[...], 0.0)


def _tc3_body(d0_ref, agg_ref, c1w_ref, c1b_ref, l1w_ref, l1b_ref, d2_ref):
    d = d0_ref[...] + agg_ref[:, :_H] + agg_ref[:, _H:]
    d = jnp.maximum(_dot(d, c1w_ref[...]) + c1b_ref[...], 0.0)
    d2_ref[...] = _dot(d, l1w_ref[...]) + l1b_ref[...]


def _tc4_body(d2_ref, agg_ref, g_ref, batch_ref,
              c2w_ref, c2b_ref, l2w_ref, l2b_ref, itw_ref, itb_ref,
              clsw_ref, clsb_ref, dec_ref, out_ref, pool_s, cnt_s):
    i = pl.program_id(0)
    d = d2_ref[...] + agg_ref[:, :_H] + agg_ref[:, _H:]
    d = jnp.maximum(_dot(d, c2w_ref[...]) + c2b_ref[...], 0.0)
    d = _dot(d, l2w_ref[...]) + l2b_ref[...]
    dec_ref[...] = _dot(d, itw_ref[...]) + itb_ref[...]

    onehot = (batch_ref[...] ==
              lax.broadcasted_iota(jnp.int32, (_B, _NB), 1)).astype(jnp.float32)
    dims = (((0,), (0,)), ((), ()))
    psum = lax.dot_general(onehot, g_ref[...], dims,
                           preferred_element_type=jnp.float32)
    csum = lax.dot_general(onehot, jnp.ones((_B, 1), jnp.float32), dims,
                           preferred_element_type=jnp.float32)

    @pl.when(i == 0)
    def _():
        pool_s[...] = jnp.zeros_like(pool_s)
        cnt_s[...] = jnp.zeros_like(cnt_s)

    pool_s[...] += psum
    cnt_s[...] += csum

    @pl.when(i == pl.num_programs(0) - 1)
    def _():
        pooled = pool_s[...] / jnp.maximum(cnt_s[...], 1.0)
        out_ref[...] = _dot(pooled, clsw_ref[...]) + clsb_ref[...]


def _full(shape):
    return pl.BlockSpec(shape, lambda i: tuple(0 for _ in shape))


def _rows(w):
    return pl.BlockSpec((_B, w), lambda i: (i, 0))


_agg_spec = pl.BlockSpec((_B, 2 * _H), lambda i: (i, 0))

_tc1 = pl.pallas_call(
    _tc1_body,
    grid=(_G,),
    in_specs=[_rows(_DIN), _full((_DIN, _H)), _full((1, _H)), _full((1, _H)),
              _full((1, _H)), _full((_H, _H)), _full((1, _H))],
    out_specs=[_rows(_H), _rows(_H)],
    out_shape=[jax.ShapeDtypeStruct((_N, _H), jnp.float32)] * 2,
)

_tc2 = pl.pallas_call(
    _tc2_body,
    grid=(_G,),
    in_specs=[_rows(_H), _agg_spec, _rows(_H), _rows(1),
              _full((_H, _H)), _full((1, _H)), _full((_H, _H)), _full((1, _H)),
              _full((_H, 2)), _full((1, 2)), _full((2, _H)), _full((1, _H))],
    out_specs=[_rows(_H), _rows(_H)],
    out_shape=[jax.ShapeDtypeStruct((_N, _H), jnp.float32)] * 2,
)

_tc3 = pl.pallas_call(
    _tc3_body,
    grid=(_G,),
    in_specs=[_rows(_H), _agg_spec,
              _full((_H, _H)), _full((1, _H)), _full((_H, _H)), _full((1, _H))],
    out_specs=[_rows(_H)],
    out_shape=[jax.ShapeDtypeStruct((_N, _H), jnp.float32)],
)

_tc4 = pl.pallas_call(
    _tc4_body,
    grid=(_G,),
    in_specs=[_rows(_H), _agg_spec, _rows(_H), _rows(1),
              _full((_H, _H)), _full((1, _H)), _full((_H, _H)), _full((1, _H)),
              _full((_H, _H)), _full((1, _H)), _full((_H, _DT)), _full((1, _DT))],
    out_specs=[_rows(_H), _full((_NB, _DT))],
    out_shape=[jax.ShapeDtypeStruct((_N, _H), jnp.float32),
               jax.ShapeDtypeStruct((_NB, _DT), jnp.float32)],
    scratch_shapes=[pltpu.VMEM((_NB, _H), jnp.float32),
                    pltpu.VMEM((_NB, 1), jnp.float32)],
)


def kernel(x, edge_index, batch, head_W, head_b, bn_g, bn_b, s0_W, s0_b,
           sc1_W, sc1_b, sl1_W, sl1_b, st_W, st_b, i0_W, i0_b,
           ic1_W, ic1_b, il1_W, il1_b, ic2_W, ic2_b, il2_W, il2_b,
           it_W, it_b, cls_W, cls_b):
    src = edge_index[0].reshape(_NW, _TPW, _CHUNK)
    dst = edge_index[1].reshape(_NW, _TPW, _CHUNK)
    zeros = jnp.zeros((_RPT, _H), jnp.float32)
    eps = jnp.asarray(_EPS)
    batch2 = batch.reshape(_N, 1)
    r = lambda v: v.reshape(1, -1)

    f, h0 = _tc1(x, head_W, r(head_b), r(bn_g), r(bn_b), s0_W, r(s0_b))
    sc_agg = _get_sc_agg()
    agg1 = sc_agg(h0, src, dst, zeros)
    d0, g = _tc2(h0, agg1, f, eps, sc1_W, r(sc1_b), sl1_W, r(sl1_b),
                 st_W, r(st_b), i0_W, r(i0_b))
    agg2 = sc_agg(d0, src, dst, zeros)
    (d2,) = _tc3(d0, agg2, ic1_W, r(ic1_b), il1_W, r(il1_b))
    agg3 = sc_agg(d2, src, dst, zeros)
    decode, out = _tc4(d2, agg3, g, batch2, ic2_W, r(ic2_b), il2_W, r(il2_b),
                       it_W, r(it_b), cls_W, r(cls_b))
    return out, decode


# speedup vs baseline: 13.7185x; 1.1125x over previous
import functools

import jax
import jax.numpy as jnp
import numpy as np
from jax import lax
from jax.experimental import pallas as pl
from jax.experimental.pallas import tpu as pltpu
from jax.experimental.pallas import tpu_sc as plsc

_N = 10000
_E = 320000
_DIN = 128
_H = 64
_DT = 4
_NB = 8

_BN_SCALE = 1.0 / (1.0 + 1e-5) ** 0.5

_EPS = np.asarray(
    jax.random.normal(jax.random.key(42), (_N, 1), dtype=jnp.float32))

_NC = 2
_NS = 16
_NW = _NC * _NS
_CHUNK = 125
_NT = _E // _CHUNK
_TPW = _NT // _NW
_GRP = 4
_NGRP = _TPW // _GRP
_NPAD = 10240
_RPT = _NPAD // _NS

@functools.cache
def _get_sc_agg():
    mesh = plsc.VectorSubcoreMesh(core_axis_name="c", subcore_axis_name="s",
                                  num_cores=_NC, num_subcores=_NS)
    return functools.partial(
        pl.kernel,
        out_type=jax.ShapeDtypeStruct((_NPAD, 2 * _H), jnp.float32),
        mesh=mesh,
        scratch_types=[
            pltpu.VMEM((_TPW, _CHUNK), jnp.int32),
            pltpu.VMEM((_TPW, _CHUNK), jnp.int32),
            pltpu.VMEM((2, _GRP, _CHUNK, _H), jnp.float32),
            pltpu.VMEM_SHARED((_NPAD, _H), jnp.float32),
            pltpu.SemaphoreType.DMA,
            pltpu.SemaphoreType.DMA,
            pltpu.SemaphoreType.DMA,
            pltpu.SemaphoreType.DMA,
        ],
        compiler_params=pltpu.CompilerParams(use_tc_tiling_on_sc=False),
    )(_sc_agg_body)


def _sc_agg_body(h_hbm, src_hbm, dst_hbm, zeros_hbm, out_hbm,
                 src_v, dst_v, rows_v, acc_sh,
                 sem_ld0, sem_ld1, sem_st0, sem_st1):
    c = lax.axis_index("c")
    s = lax.axis_index("s")
    wid = s * _NC + c
    sem_ld = (sem_ld0, sem_ld1)
    sem_st = (sem_st0, sem_st1)

    d_src = pltpu.async_copy(src_hbm.at[wid], src_v, sem_ld0)
    d_dst = pltpu.async_copy(dst_hbm.at[wid], dst_v, sem_ld0)
    d_zero = pltpu.async_copy(zeros_hbm, acc_sh.at[pl.ds(s * _RPT, _RPT)],
                              sem_st1)
    d_src.wait()
    d_dst.wait()

    def fire_gathers(g, b):
        for j in range(_GRP):
            pltpu.async_copy(h_hbm.at[src_v.at[g * _GRP + j]],
                             rows_v.at[b, j], sem_ld[b])

    def drain_gathers(g, b):
        for j in range(_GRP):
            pltpu.make_async_copy(h_hbm.at[src_v.at[g * _GRP + j]],
                                  rows_v.at[b, j], sem_ld[b]).wait()

    def fire_scatters(g, b):
        for j in range(_GRP):
            pltpu.async_copy(rows_v.at[b, j], acc_sh.at[dst_v.at[g * _GRP + j]],
                             sem_st[b], add=True)

    def drain_scatters(g, b):
        for j in range(_GRP):
            pltpu.make_async_copy(rows_v.at[b, j],
                                  acc_sh.at[dst_v.at[g * _GRP + j]],
                                  sem_st[b]).wait()

    fire_gathers(0, 0)
    d_zero.wait()
    plsc.subcore_barrier()

    def pair(i, carry):
        g0 = 2 * i
        g1 = g0 + 1

        @pl.when(i > 0)
        def _():
            drain_scatters(g1 - 2, 1)
        fire_gathers(g1, 1)
        drain_gathers(g0, 0)
        fire_scatters(g0, 0)
        drain_gathers(g1, 1)
        fire_scatters(g1, 1)

        @pl.when(i < _NGRP // 2 - 1)
        def _():
            drain_scatters(g0, 0)
            fire_gathers(g0 + 2, 0)
        return carry

    lax.fori_loop(0, _NGRP // 2, pair, 0)
    drain_scatters(_NGRP - 2, 0)
    drain_scatters(_NGRP - 1, 1)
    plsc.subcore_barrier()
    pltpu.sync_copy(acc_sh.at[pl.ds(s * _RPT, _RPT)],
                    out_hbm.at[pl.ds(s * _RPT, _RPT), pl.ds(c * _H, _H)])


_B = 1000
_G = _N // _B


def _dot(a, b):
    return jnp.dot(a, b, preferred_element_type=jnp.float32)


def _tc1_body(x_ref, hw_ref, hb_ref, g_ref, b_ref, s0w_ref, s0b_ref,
              f_ref, h0_ref):
    y = _dot(x_ref[...], hw_ref[...]) + hb_ref[...]
    f = jnp.maximum(y * (g_ref[...] * _BN_SCALE) + b_ref[...], 0.0)
    f_ref[...] = f
    h0_ref[...] = jnp.maximum(_dot(f, s0w_ref[...]) + s0b_ref[...], 0.0)


def _tc2_body(h0_ref, agg_ref, f_ref, eps_ref,
              c1w_ref, c1b_ref, l1w_ref, l1b_ref, stw_ref, stb_ref,
              i0w_ref, i0b_ref, d0_ref, g_out_ref):
    h = h0_ref[...] + agg_ref[:, :_H] + agg_ref[:, _H:]
    h = jnp.maximum(_dot(h, c1w_ref[...]) + c1b_ref[...], 0.0)
    h = _dot(h, l1w_ref[...]) + l1b_ref[...]
    enc = _dot(h, stw_ref[...]) + stb_ref[...]
    loc = enc[:, 0:1]
    logvar = jnp.clip(enc[:, 1:2], -20.0, 20.0)
    gate = loc + jnp.exp(0.5 * logvar) * eps_ref[...]
    g_out_ref[...] = f_ref[...] * gate
    d0_ref[...] = jnp.maximum(_dot(enc, i0w_ref[...]) + i0b_ref[...], 0.0)


def _tc3_body(d0_ref, agg_ref, c1w_ref, c1b_ref, l1w_ref, l1b_ref, d2_ref):
    d = d0_ref[...] + agg_ref[:, :_H] + agg_ref[:, _H:]
    d = jnp.maximum(_dot(d, c1w_ref[...]) + c1b_ref[...], 0.0)
    d2_ref[...] = _dot(d, l1w_ref[...]) + l1b_ref[...]


def _tc4_body(d2_ref, agg_ref, g_ref, batch_ref,
              c2w_ref, c2b_ref, l2w_ref, l2b_ref, itw_ref, itb_ref,
              clsw_ref, clsb_ref, dec_ref, out_ref, pool_s, cnt_s):
    i = pl.program_id(0)
    d = d2_ref[...] + agg_ref[:, :_H] + agg_ref[:, _H:]
    d = jnp.maximum(_dot(d, c2w_ref[...]) + c2b_ref[...], 0.0)
    d = _dot(d, l2w_ref[...]) + l2b_ref[...]
    dec_ref[...] = _dot(d, itw_ref[...]) + itb_ref[...]

    onehot = (batch_ref[...] ==
              lax.broadcasted_iota(jnp.int32, (_B, _NB), 1)).astype(jnp.float32)
    dims = (((0,), (0,)), ((), ()))
    psum = lax.dot_general(onehot, g_ref[...], dims,
                           preferred_element_type=jnp.float32)
    csum = lax.dot_general(onehot, jnp.ones((_B, 1), jnp.float32), dims,
                           preferred_element_type=jnp.float32)

    @pl.when(i == 0)
    def _():
        pool_s[...] = jnp.zeros_like(pool_s)
        cnt_s[...] = jnp.zeros_like(cnt_s)

    pool_s[...] += psum
    cnt_s[...] += csum

    @pl.when(i == pl.num_programs(0) - 1)
    def _():
        pooled = pool_s[...] / jnp.maximum(cnt_s[...], 1.0)
        out_ref[...] = _dot(pooled, clsw_ref[...]) + clsb_ref[...]


def _full(shape):
    return pl.BlockSpec(shape, lambda i: tuple(0 for _ in shape))


def _rows(w):
    return pl.BlockSpec((_B, w), lambda i: (i, 0))


_agg_spec = pl.BlockSpec((_B, 2 * _H), lambda i: (i, 0))

_tc1 = pl.pallas_call(
    _tc1_body,
    grid=(_G,),
    in_specs=[_rows(_DIN), _full((_DIN, _H)), _full((1, _H)), _full((1, _H)),
              _full((1, _H)), _full((_H, _H)), _full((1, _H))],
    out_specs=[_rows(_H), _rows(_H)],
    out_shape=[jax.ShapeDtypeStruct((_N, _H), jnp.float32)] * 2,
)

_tc2 = pl.pallas_call(
    _tc2_body,
    grid=(_G,),
    in_specs=[_rows(_H), _agg_spec, _rows(_H), _rows(1),
              _full((_H, _H)), _full((1, _H)), _full((_H, _H)), _full((1, _H)),
              _full((_H, 2)), _full((1, 2)), _full((2, _H)), _full((1, _H))],
    out_specs=[_rows(_H), _rows(_H)],
    out_shape=[jax.ShapeDtypeStruct((_N, _H), jnp.float32)] * 2,
)

_tc3 = pl.pallas_call(
    _tc3_body,
    grid=(_G,),
    in_specs=[_rows(_H), _agg_spec,
              _full((_H, _H)), _full((1, _H)), _full((_H, _H)), _full((1, _H))],
    out_specs=[_rows(_H)],
    out_shape=[jax.ShapeDtypeStruct((_N, _H), jnp.float32)],
)

_tc4 = pl.pallas_call(
    _tc4_body,
    grid=(_G,),
    in_specs=[_rows(_H), _agg_spec, _rows(_H), _rows(1),
              _full((_H, _H)), _full((1, _H)), _full((_H, _H)), _full((1, _H)),
              _full((_H, _H)), _full((1, _H)), _full((_H, _DT)), _full((1, _DT))],
    out_specs=[_rows(_H), _full((_NB, _DT))],
    out_shape=[jax.ShapeDtypeStruct((_N, _H), jnp.float32),
               jax.ShapeDtypeStruct((_NB, _DT), jnp.float32)],
    scratch_shapes=[pltpu.VMEM((_NB, _H), jnp.float32),
                    pltpu.VMEM((_NB, 1), jnp.float32)],
)


def kernel(x, edge_index, batch, head_W, head_b, bn_g, bn_b, s0_W, s0_b,
           sc1_W, sc1_b, sl1_W, sl1_b, st_W, st_b, i0_W, i0_b,
           ic1_W, ic1_b, il1_W, il1_b, ic2_W, ic2_b, il2_W, il2_b,
           it_W, it_b, cls_W, cls_b):
    src = edge_index[0].reshape(_NW, _TPW, _CHUNK)
    dst = edge_index[1].reshape(_NW, _TPW, _CHUNK)
    zeros = jnp.zeros((_RPT, _H), jnp.float32)
    eps = jnp.asarray(_EPS)
    batch2 = batch.reshape(_N, 1)
    r = lambda v: v.reshape(1, -1)

    f, h0 = _tc1(x, head_W, r(head_b), r(bn_g), r(bn_b), s0_W, r(s0_b))
    sc_agg = _get_sc_agg()
    agg1 = sc_agg(h0, src, dst, zeros)
    d0, g = _tc2(h0, agg1, f, eps, sc1_W, r(sc1_b), sl1_W, r(sl1_b),
                 st_W, r(st_b), i0_W, r(i0_b))
    agg2 = sc_agg(d0, src, dst, zeros)
    (d2,) = _tc3(d0, agg2, ic1_W, r(ic1_b), il1_W, r(il1_b))
    agg3 = sc_agg(d2, src, dst, zeros)
    decode, out = _tc4(d2, agg3, g, batch2, ic2_W, r(ic2_b), il2_W, r(il2_b),
                       it_W, r(it_b), cls_W, r(cls_b))
    return out, decode


# speedup vs baseline: 14.2301x; 1.0373x over previous
import functools

import jax
import jax.numpy as jnp
import numpy as np
from jax import lax
from jax.experimental import pallas as pl
from jax.experimental.pallas import tpu as pltpu
from jax.experimental.pallas import tpu_sc as plsc

_N = 10000
_E = 320000
_DIN = 128
_H = 64
_DT = 4
_NB = 8

_BN_SCALE = 1.0 / (1.0 + 1e-5) ** 0.5

_EPS = np.asarray(
    jax.random.normal(jax.random.key(42), (_N, 1), dtype=jnp.float32))

_NC = 2
_NS = 16
_NW = _NC * _NS
_CHUNK = 125
_NT = _E // _CHUNK
_TPW = _NT // _NW
_GRP = 4
_NGRP = _TPW // _GRP
_NPAD = 10240
_RPT = _NPAD // _NS

@functools.cache
def _get_sc_agg():
    mesh = plsc.VectorSubcoreMesh(core_axis_name="c", subcore_axis_name="s",
                                  num_cores=_NC, num_subcores=_NS)
    return functools.partial(
        pl.kernel,
        out_type=jax.ShapeDtypeStruct((_NPAD, 2 * _H), jnp.float32),
        mesh=mesh,
        scratch_types=[
            pltpu.VMEM((_TPW, _CHUNK), jnp.int32),
            pltpu.VMEM((_TPW, _CHUNK), jnp.int32),
            pltpu.VMEM((2, _GRP, _CHUNK, _H), jnp.float32),
            pltpu.VMEM_SHARED((_NPAD, _H), jnp.float32),
            pltpu.SemaphoreType.DMA,
            pltpu.SemaphoreType.DMA,
            pltpu.SemaphoreType.DMA,
            pltpu.SemaphoreType.DMA,
        ],
        compiler_params=pltpu.CompilerParams(use_tc_tiling_on_sc=False),
    )(_sc_agg_body)


def _sc_agg_body(h_hbm, src_hbm, dst_hbm, zeros_hbm, out_hbm,
                 src_v, dst_v, rows_v, acc_sh,
                 sem_ld0, sem_ld1, sem_st0, sem_st1):
    c = lax.axis_index("c")
    s = lax.axis_index("s")
    wid = s * _NC + c
    sem_ld = (sem_ld0, sem_ld1)
    sem_st = (sem_st0, sem_st1)

    d_src = pltpu.async_copy(src_hbm.at[wid], src_v, sem_ld0)
    d_dst = pltpu.async_copy(dst_hbm.at[wid], dst_v, sem_ld0)
    d_zero = pltpu.async_copy(zeros_hbm, acc_sh.at[pl.ds(s * _RPT, _RPT)],
                              sem_st1)
    d_src.wait()
    d_dst.wait()

    def fire_gathers(g, b):
        for j in range(_GRP):
            pltpu.async_copy(h_hbm.at[src_v.at[g * _GRP + j]],
                             rows_v.at[b, j], sem_ld[b])

    def drain_gathers(g, b):
        for j in range(_GRP):
            pltpu.make_async_copy(h_hbm.at[src_v.at[g * _GRP + j]],
                                  rows_v.at[b, j], sem_ld[b]).wait()

    def fire_scatters(g, b):
        for j in range(_GRP):
            pltpu.async_copy(rows_v.at[b, j], acc_sh.at[dst_v.at[g * _GRP + j]],
                             sem_st[b], add=True)

    def drain_scatters(g, b):
        for j in range(_GRP):
            pltpu.make_async_copy(rows_v.at[b, j],
                                  acc_sh.at[dst_v.at[g * _GRP + j]],
                                  sem_st[b]).wait()

    fire_gathers(0, 0)
    d_zero.wait()
    plsc.subcore_barrier()

    def pair(i, carry):
        g0 = 2 * i
        g1 = g0 + 1

        @pl.when(i > 0)
        def _():
            drain_scatters(g1 - 2, 1)
        fire_gathers(g1, 1)
        drain_gathers(g0, 0)
        fire_scatters(g0, 0)
        drain_gathers(g1, 1)
        fire_scatters(g1, 1)

        @pl.when(i < _NGRP // 2 - 1)
        def _():
            drain_scatters(g0, 0)
            fire_gathers(g0 + 2, 0)
        return carry

    lax.fori_loop(0, _NGRP // 2, pair, 0)
    drain_scatters(_NGRP - 2, 0)
    drain_scatters(_NGRP - 1, 1)
    plsc.subcore_barrier()
    pltpu.sync_copy(acc_sh.at[pl.ds(s * _RPT, _RPT)],
                    out_hbm.at[pl.ds(s * _RPT, _RPT), pl.ds(c * _H, _H)])


_B = 2000
_G = _N // _B


def _dot(a, b):
    return jnp.dot(a, b, preferred_element_type=jnp.float32)


def _tc1_body(x_ref, hw_ref, hb_ref, g_ref, b_ref, s0w_ref, s0b_ref,
              f_ref, h0_ref):
    y = _dot(x_ref[...], hw_ref[...]) + hb_ref[...]
    f = jnp.maximum(y * (g_ref[...] * _BN_SCALE) + b_ref[...], 0.0)
    f_ref[...] = f
    h0_ref[...] = jnp.maximum(_dot(f, s0w_ref[...]) + s0b_ref[...], 0.0)


def _tc2_body(h0_ref, agg_ref, f_ref, eps_ref,
              c1w_ref, c1b_ref, l1w_ref, l1b_ref, stw_ref, stb_ref,
              i0w_ref, i0b_ref, d0_ref, g_out_ref):
    h = h0_ref[...] + agg_ref[:, :_H] + agg_ref[:, _H:]
    h = jnp.maximum(_dot(h, c1w_ref[...]) + c1b_ref[...], 0.0)
    h = _dot(h, l1w_ref[...]) + l1b_ref[...]
    enc = _dot(h, stw_ref[...]) + stb_ref[...]
    loc = enc[:, 0:1]
    logvar = jnp.clip(enc[:, 1:2], -20.0, 20.0)
    gate = loc + jnp.exp(0.5 * logvar) * eps_ref[...]
    g_out_ref[...] = f_ref[...] * gate
    d0_ref[...] = jnp.maximum(_dot(enc, i0w_ref[...]) + i0b_ref[...], 0.0)


def _tc3_body(d0_ref, agg_ref, c1w_ref, c1b_ref, l1w_ref, l1b_ref, d2_ref):
    d = d0_ref[...] + agg_ref[:, :_H] + agg_ref[:, _H:]
    d = jnp.maximum(_dot(d, c1w_ref[...]) + c1b_ref[...], 0.0)
    d2_ref[...] = _dot(d, l1w_ref[...]) + l1b_ref[...]


def _tc4_body(d2_ref, agg_ref, g_ref, batch_ref,
              c2w_ref, c2b_ref, l2w_ref, l2b_ref, itw_ref, itb_ref,
              clsw_ref, clsb_ref, dec_ref, out_ref, pool_s, cnt_s):
    i = pl.program_id(0)
    d = d2_ref[...] + agg_ref[:, :_H] + agg_ref[:, _H:]
    d = jnp.maximum(_dot(d, c2w_ref[...]) + c2b_ref[...], 0.0)
    d = _dot(d, l2w_ref[...]) + l2b_ref[...]
    dec_ref[...] = _dot(d, itw_ref[...]) + itb_ref[...]

    onehot = (batch_ref[...] ==
              lax.broadcasted_iota(jnp.int32, (_B, _NB), 1)).astype(jnp.float32)
    dims = (((0,), (0,)), ((), ()))
    psum = lax.dot_general(onehot, g_ref[...], dims,
                           preferred_element_type=jnp.float32)
    csum = lax.dot_general(onehot, jnp.ones((_B, 1), jnp.float32), dims,
                           preferred_element_type=jnp.float32)

    @pl.when(i == 0)
    def _():
        pool_s[...] = jnp.zeros_like(pool_s)
        cnt_s[...] = jnp.zeros_like(cnt_s)

    pool_s[...] += psum
    cnt_s[...] += csum

    @pl.when(i == pl.num_programs(0) - 1)
    def _():
        pooled = pool_s[...] / jnp.maximum(cnt_s[...], 1.0)
        out_ref[...] = _dot(pooled, clsw_ref[...]) + clsb_ref[...]


def _full(shape):
    return pl.BlockSpec(shape, lambda i: tuple(0 for _ in shape))


def _rows(w):
    return pl.BlockSpec((_B, w), lambda i: (i, 0))


_agg_spec = pl.BlockSpec((_B, 2 * _H), lambda i: (i, 0))

_tc1 = pl.pallas_call(
    _tc1_body,
    grid=(_G,),
    in_specs=[_rows(_DIN), _full((_DIN, _H)), _full((1, _H)), _full((1, _H)),
              _full((1, _H)), _full((_H, _H)), _full((1, _H))],
    out_specs=[_rows(_H), _rows(_H)],
    out_shape=[jax.ShapeDtypeStruct((_N, _H), jnp.float32)] * 2,
)

_tc2 = pl.pallas_call(
    _tc2_body,
    grid=(_G,),
    in_specs=[_rows(_H), _agg_spec, _rows(_H), _rows(1),
              _full((_H, _H)), _full((1, _H)), _full((_H, _H)), _full((1, _H)),
              _full((_H, 2)), _full((1, 2)), _full((2, _H)), _full((1, _H))],
    out_specs=[_rows(_H), _rows(_H)],
    out_shape=[jax.ShapeDtypeStruct((_N, _H), jnp.float32)] * 2,
)

_tc3 = pl.pallas_call(
    _tc3_body,
    grid=(_G,),
    in_specs=[_rows(_H), _agg_spec,
              _full((_H, _H)), _full((1, _H)), _full((_H, _H)), _full((1, _H))],
    out_specs=[_rows(_H)],
    out_shape=[jax.ShapeDtypeStruct((_N, _H), jnp.float32)],
)

_tc4 = pl.pallas_call(
    _tc4_body,
    grid=(_G,),
    in_specs=[_rows(_H), _agg_spec, _rows(_H), _rows(1),
              _full((_H, _H)), _full((1, _H)), _full((_H, _H)), _full((1, _H)),
              _full((_H, _H)), _full((1, _H)), _full((_H, _DT)), _full((1, _DT))],
    out_specs=[_rows(_H), _full((_NB, _DT))],
    out_shape=[jax.ShapeDtypeStruct((_N, _H), jnp.float32),
               jax.ShapeDtypeStruct((_NB, _DT), jnp.float32)],
    scratch_shapes=[pltpu.VMEM((_NB, _H), jnp.float32),
                    pltpu.VMEM((_NB, 1), jnp.float32)],
)


def kernel(x, edge_index, batch, head_W, head_b, bn_g, bn_b, s0_W, s0_b,
           sc1_W, sc1_b, sl1_W, sl1_b, st_W, st_b, i0_W, i0_b,
           ic1_W, ic1_b, il1_W, il1_b, ic2_W, ic2_b, il2_W, il2_b,
           it_W, it_b, cls_W, cls_b):
    src = edge_index[0].reshape(_NW, _TPW, _CHUNK)
    dst = edge_index[1].reshape(_NW, _TPW, _CHUNK)
    zeros = jnp.zeros((_RPT, _H), jnp.float32)
    eps = jnp.asarray(_EPS)
    batch2 = batch.reshape(_N, 1)
    r = lambda v: v.reshape(1, -1)

    f, h0 = _tc1(x, head_W, r(head_b), r(bn_g), r(bn_b), s0_W, r(s0_b))
    sc_agg = _get_sc_agg()
    agg1 = sc_agg(h0, src, dst, zeros)
    d0, g = _tc2(h0, agg1, f, eps, sc1_W, r(sc1_b), sl1_W, r(sl1_b),
                 st_W, r(st_b), i0_W, r(i0_b))
    agg2 = sc_agg(d0, src, dst, zeros)
    (d2,) = _tc3(d0, agg2, ic1_W, r(ic1_b), il1_W, r(il1_b))
    agg3 = sc_agg(d2, src, dst, zeros)
    decode, out = _tc4(d2, agg3, g, batch2, ic2_W, r(ic2_b), il2_W, r(il2_b),
                       it_W, r(it_b), cls_W, r(cls_b))
    return out, decode


# speedup vs baseline: 14.4441x; 1.0150x over previous
import functools

import jax
import jax.numpy as jnp
import numpy as np
from jax import lax
from jax.experimental import pallas as pl
from jax.experimental.pallas import tpu as pltpu
from jax.experimental.pallas import tpu_sc as plsc

_N = 10000
_E = 320000
_DIN = 128
_H = 64
_DT = 4
_NB = 8

_BN_SCALE = 1.0 / (1.0 + 1e-5) ** 0.5

_EPS = np.asarray(
    jax.random.normal(jax.random.key(42), (_N, 1), dtype=jnp.float32))

_NC = 2
_NS = 16
_NW = _NC * _NS
_CHUNK = 125
_NT = _E // _CHUNK
_TPW = _NT // _NW
_GRP = 4
_NGRP = _TPW // _GRP
_NPAD = 10240
_RPT = _NPAD // _NS

@functools.cache
def _get_sc_agg():
    mesh = plsc.VectorSubcoreMesh(core_axis_name="c", subcore_axis_name="s",
                                  num_cores=_NC, num_subcores=_NS)
    return functools.partial(
        pl.kernel,
        out_type=jax.ShapeDtypeStruct((_NPAD, 2 * _H), jnp.float32),
        mesh=mesh,
        scratch_types=[
            pltpu.VMEM((_TPW, _CHUNK), jnp.int32),
            pltpu.VMEM((_TPW, _CHUNK), jnp.int32),
            pltpu.VMEM((2, _GRP, _CHUNK, _H), jnp.float32),
            pltpu.VMEM_SHARED((_NPAD, _H), jnp.float32),
            pltpu.SemaphoreType.DMA,
            pltpu.SemaphoreType.DMA,
            pltpu.SemaphoreType.DMA,
            pltpu.SemaphoreType.DMA,
        ],
        compiler_params=pltpu.CompilerParams(use_tc_tiling_on_sc=False),
    )(_sc_agg_body)


def _sc_agg_body(h_hbm, src_hbm, dst_hbm, zeros_hbm, out_hbm,
                 src_v, dst_v, rows_v, acc_sh,
                 sem_ld0, sem_ld1, sem_st0, sem_st1):
    c = lax.axis_index("c")
    s = lax.axis_index("s")
    wid = s * _NC + c
    sem_ld = (sem_ld0, sem_ld1)
    sem_st = (sem_st0, sem_st1)

    d_src = pltpu.async_copy(src_hbm.at[wid], src_v, sem_ld0)
    d_dst = pltpu.async_copy(dst_hbm.at[wid], dst_v, sem_ld0)
    d_zero = pltpu.async_copy(zeros_hbm, acc_sh.at[pl.ds(s * _RPT, _RPT)],
                              sem_st1)
    d_src.wait()
    d_dst.wait()

    def fire_gathers(g, b):
        for j in range(_GRP):
            pltpu.async_copy(h_hbm.at[src_v.at[g * _GRP + j]],
                             rows_v.at[b, j], sem_ld[b])

    def drain_gathers(g, b):
        for j in range(_GRP):
            pltpu.make_async_copy(h_hbm.at[src_v.at[g * _GRP + j]],
                                  rows_v.at[b, j], sem_ld[b]).wait()

    def fire_scatters(g, b):
        for j in range(_GRP):
            pltpu.async_copy(rows_v.at[b, j], acc_sh.at[dst_v.at[g * _GRP + j]],
                             sem_st[b], add=True)

    def drain_scatters(g, b):
        for j in range(_GRP):
            pltpu.make_async_copy(rows_v.at[b, j],
                                  acc_sh.at[dst_v.at[g * _GRP + j]],
                                  sem_st[b]).wait()

    fire_gathers(0, 0)
    d_zero.wait()
    plsc.subcore_barrier()

    def pair(i, carry):
        g0 = 2 * i
        g1 = g0 + 1

        @pl.when(i > 0)
        def _():
            drain_scatters(g1 - 2, 1)
        fire_gathers(g1, 1)
        drain_gathers(g0, 0)
        fire_scatters(g0, 0)
        drain_gathers(g1, 1)
        fire_scatters(g1, 1)

        @pl.when(i < _NGRP // 2 - 1)
        def _():
            drain_scatters(g0, 0)
            fire_gathers(g0 + 2, 0)
        return carry

    lax.fori_loop(0, _NGRP // 2, pair, 0)
    drain_scatters(_NGRP - 2, 0)
    drain_scatters(_NGRP - 1, 1)
    plsc.subcore_barrier()
    pltpu.sync_copy(acc_sh.at[pl.ds(s * _RPT, _RPT)],
                    out_hbm.at[pl.ds(s * _RPT, _RPT), pl.ds(c * _H, _H)])


_B = 5000
_G = _N // _B


def _dot(a, b):
    return jnp.dot(a, b, preferred_element_type=jnp.float32)


def _tc1_body(x_ref, hw_ref, hb_ref, g_ref, b_ref, s0w_ref, s0b_ref,
              f_ref, h0_ref):
    y = _dot(x_ref[...], hw_ref[...]) + hb_ref[...]
    f = jnp.maximum(y * (g_ref[...] * _BN_SCALE) + b_ref[...], 0.0)
    f_ref[...] = f
    h0_ref[...] = jnp.maximum(_dot(f, s0w_ref[...]) + s0b_ref[...], 0.0)


def _tc2_body(h0_ref, agg_ref, f_ref, eps_ref,
              c1w_ref, c1b_ref, l1w_ref, l1b_ref, stw_ref, stb_ref,
              i0w_ref, i0b_ref, d0_ref, g_out_ref):
    h = h0_ref[...] + agg_ref[:, :_H] + agg_ref[:, _H:]
    h = jnp.maximum(_dot(h, c1w_ref[...]) + c1b_ref[...], 0.0)
    h = _dot(h, l1w_ref[...]) + l1b_ref[...]
    enc = _dot(h, stw_ref[...]) + stb_ref[...]
    loc = enc[:, 0:1]
    logvar = jnp.clip(enc[:, 1:2], -20.0, 20.0)
    gate = loc + jnp.exp(0.5 * logvar) * eps_ref[...]
    g_out_ref[...] = f_ref[...] * gate
    d0_ref[...] = jnp.maximum(_dot(enc, i0w_ref[...]) + i0b_ref[...], 0.0)


def _tc3_body(d0_ref, agg_ref, c1w_ref, c1b_ref, l1w_ref, l1b_ref, d2_ref):
    d = d0_ref[...] + agg_ref[:, :_H] + agg_ref[:, _H:]
    d = jnp.maximum(_dot(d, c1w_ref[...]) + c1b_ref[...], 0.0)
    d2_ref[...] = _dot(d, l1w_ref[...]) + l1b_ref[...]


def _tc4_body(d2_ref, agg_ref, g_ref, batch_ref,
              c2w_ref, c2b_ref, l2w_ref, l2b_ref, itw_ref, itb_ref,
              clsw_ref, clsb_ref, dec_ref, out_ref, pool_s, cnt_s):
    i = pl.program_id(0)
    d = d2_ref[...] + agg_ref[:, :_H] + agg_ref[:, _H:]
    d = jnp.maximum(_dot(d, c2w_ref[...]) + c2b_ref[...], 0.0)
    d = _dot(d, l2w_ref[...]) + l2b_ref[...]
    dec_ref[...] = _dot(d, itw_ref[...]) + itb_ref[...]

    onehot = (batch_ref[...] ==
              lax.broadcasted_iota(jnp.int32, (_B, _NB), 1)).astype(jnp.float32)
    dims = (((0,), (0,)), ((), ()))
    psum = lax.dot_general(onehot, g_ref[...], dims,
                           preferred_element_type=jnp.float32)
    csum = lax.dot_general(onehot, jnp.ones((_B, 1), jnp.float32), dims,
                           preferred_element_type=jnp.float32)

    @pl.when(i == 0)
    def _():
        pool_s[...] = jnp.zeros_like(pool_s)
        cnt_s[...] = jnp.zeros_like(cnt_s)

    pool_s[...] += psum
    cnt_s[...] += csum

    @pl.when(i == pl.num_programs(0) - 1)
    def _():
        pooled = pool_s[...] / jnp.maximum(cnt_s[...], 1.0)
        out_ref[...] = _dot(pooled, clsw_ref[...]) + clsb_ref[...]


def _full(shape):
    return pl.BlockSpec(shape, lambda i: tuple(0 for _ in shape))


def _rows(w):
    return pl.BlockSpec((_B, w), lambda i: (i, 0))


_agg_spec = pl.BlockSpec((_B, 2 * _H), lambda i: (i, 0))

_tc1 = pl.pallas_call(
    _tc1_body,
    grid=(_G,),
    in_specs=[_rows(_DIN), _full((_DIN, _H)), _full((1, _H)), _full((1, _H)),
              _full((1, _H)), _full((_H, _H)), _full((1, _H))],
    out_specs=[_rows(_H), _rows(_H)],
    out_shape=[jax.ShapeDtypeStruct((_N, _H), jnp.float32)] * 2,
)

_tc2 = pl.pallas_call(
    _tc2_body,
    grid=(_G,),
    in_specs=[_rows(_H), _agg_spec, _rows(_H), _rows(1),
              _full((_H, _H)), _full((1, _H)), _full((_H, _H)), _full((1, _H)),
              _full((_H, 2)), _full((1, 2)), _full((2, _H)), _full((1, _H))],
    out_specs=[_rows(_H), _rows(_H)],
    out_shape=[jax.ShapeDtypeStruct((_N, _H), jnp.float32)] * 2,
)

_tc3 = pl.pallas_call(
    _tc3_body,
    grid=(_G,),
    in_specs=[_rows(_H), _agg_spec,
              _full((_H, _H)), _full((1, _H)), _full((_H, _H)), _full((1, _H))],
    out_specs=[_rows(_H)],
    out_shape=[jax.ShapeDtypeStruct((_N, _H), jnp.float32)],
)

_tc4 = pl.pallas_call(
    _tc4_body,
    grid=(_G,),
    in_specs=[_rows(_H), _agg_spec, _rows(_H), _rows(1),
              _full((_H, _H)), _full((1, _H)), _full((_H, _H)), _full((1, _H)),
              _full((_H, _H)), _full((1, _H)), _full((_H, _DT)), _full((1, _DT))],
    out_specs=[_rows(_H), _full((_NB, _DT))],
    out_shape=[jax.ShapeDtypeStruct((_N, _H), jnp.float32),
               jax.ShapeDtypeStruct((_NB, _DT), jnp.float32)],
    scratch_shapes=[pltpu.VMEM((_NB, _H), jnp.float32),
                    pltpu.VMEM((_NB, 1), jnp.float32)],
)


def kernel(x, edge_index, batch, head_W, head_b, bn_g, bn_b, s0_W, s0_b,
           sc1_W, sc1_b, sl1_W, sl1_b, st_W, st_b, i0_W, i0_b,
           ic1_W, ic1_b, il1_W, il1_b, ic2_W, ic2_b, il2_W, il2_b,
           it_W, it_b, cls_W, cls_b):
    src = edge_index[0].reshape(_NW, _TPW, _CHUNK)
    dst = edge_index[1].reshape(_NW, _TPW, _CHUNK)
    zeros = jnp.zeros((_RPT, _H), jnp.float32)
    eps = jnp.asarray(_EPS)
    batch2 = batch.reshape(_N, 1)
    r = lambda v: v.reshape(1, -1)

    f, h0 = _tc1(x, head_W, r(head_b), r(bn_g), r(bn_b), s0_W, r(s0_b))
    sc_agg = _get_sc_agg()
    agg1 = sc_agg(h0, src, dst, zeros)
    d0, g = _tc2(h0, agg1, f, eps, sc1_W, r(sc1_b), sl1_W, r(sl1_b),
                 st_W, r(st_b), i0_W, r(i0_b))
    agg2 = sc_agg(d0, src, dst, zeros)
    (d2,) = _tc3(d0, agg2, ic1_W, r(ic1_b), il1_W, r(il1_b))
    agg3 = sc_agg(d2, src, dst, zeros)
    decode, out = _tc4(d2, agg3, g, batch2, ic2_W, r(ic2_b), il2_W, r(il2_b),
                       it_W, r(it_b), cls_W, r(cls_b))
    return out, decode
